# baseline (speedup 1.0000x reference)
import jax
import jax.numpy as jnp
from jax.experimental import pallas as pl


_BLK = 2000


def _body(gt_ref, pts_ref, out_ref):
    cx = gt_ref[0:1, :]
    cy = gt_ref[1:2, :]
    w = gt_ref[2:3, :]
    h = gt_ref[3:4, :]
    ang = gt_ref[4:5, :]
    cos = jnp.cos(ang)
    sin = jnp.sin(ang)
    ia = 1.0 / ((w * 0.5) ** 2)
    ib = 1.0 / ((h * 0.5) ** 2)
    a_c = cos * cos * ia + sin * sin * ib
    c_c = sin * sin * ia + cos * cos * ib
    b_c = 2.0 * cos * sin * (ia - ib)
    l1 = -(2.0 * a_c * cx + b_c * cy)
    l2 = -(2.0 * c_c * cy + b_c * cx)
    k_c = a_c * cx * cx + c_c * cy * cy + b_c * cx * cy
    g = jnp.concatenate([a_c, c_c, b_c, l1, l2, k_c], axis=0)

    px = pts_ref[:, 0:1]
    py = pts_ref[:, 1:2]
    ones = jnp.ones_like(px)
    f = jnp.concatenate([px * px, py * py, px * py, px, py, ones], axis=1)
    out_ref[...] = jnp.dot(f, g, preferred_element_type=jnp.float32,
                           precision=jax.lax.Precision.HIGHEST)


def kernel(gt_bboxes, points):
    num_gts = gt_bboxes.shape[0]
    num_points = points.shape[0]
    gt_t = gt_bboxes.T
    grid = (num_points // _BLK,)
    return pl.pallas_call(
        _body,
        grid=grid,
        in_specs=[
            pl.BlockSpec((5, num_gts), lambda i: (0, 0)),
            pl.BlockSpec((_BLK, 2), lambda i: (i, 0)),
        ],
        out_specs=pl.BlockSpec((_BLK, num_gts), lambda i: (i, 0)),
        out_shape=jax.ShapeDtypeStruct((num_points, num_gts), jnp.float32),
    )(gt_t, points)


# speedup vs baseline: 1.2035x; 1.2035x over previous
import jax
import jax.numpy as jnp
from jax.experimental import pallas as pl


_BLK = 1000


def _body(gt_ref, pts_ref, out_ref):
    cx = gt_ref[0:1, :]
    cy = gt_ref[1:2, :]
    w = gt_ref[2:3, :]
    h = gt_ref[3:4, :]
    ang = gt_ref[4:5, :]
    cos = jnp.cos(ang)
    sin = jnp.sin(ang)
    inv_a = 2.0 / w
    inv_b = 2.0 / h
    ca = cos * inv_a
    sa = sin * inv_a
    cb = cos * inv_b
    sb = sin * inv_b
    px = pts_ref[:, 0:1]
    py = pts_ref[:, 1:2]
    dx = px - cx
    dy = py - cy
    ox = ca * dx + sa * dy
    oy = cb * dy - sb * dx
    out_ref[...] = ox * ox + oy * oy


def kernel(gt_bboxes, points):
    num_gts = gt_bboxes.shape[0]
    num_points = points.shape[0]
    gt_t = gt_bboxes.T
    grid = (num_points // _BLK,)
    return pl.pallas_call(
        _body,
        grid=grid,
        in_specs=[
            pl.BlockSpec((5, num_gts), lambda i: (0, 0)),
            pl.BlockSpec((_BLK, 2), lambda i: (i, 0)),
        ],
        out_specs=pl.BlockSpec((_BLK, num_gts), lambda i: (i, 0)),
        out_shape=jax.ShapeDtypeStruct((num_points, num_gts), jnp.float32),
    )(gt_t, points)


# speedup vs baseline: 1.3145x; 1.0922x over previous
import jax
import jax.numpy as jnp
from jax.experimental import pallas as pl


_BLK = 4000


def _body(gt_ref, pts_ref, out_ref):
    cx = gt_ref[0:1, :]
    cy = gt_ref[1:2, :]
    w = gt_ref[2:3, :]
    h = gt_ref[3:4, :]
    ang = gt_ref[4:5, :]
    cos = jnp.cos(ang)
    sin = jnp.sin(ang)
    inv_a = 2.0 / w
    inv_b = 2.0 / h
    ca = cos * inv_a
    sa = sin * inv_a
    cb = cos * inv_b
    sb = sin * inv_b
    px = pts_ref[:, 0:1]
    py = pts_ref[:, 1:2]
    dx = px - cx
    dy = py - cy
    ox = ca * dx + sa * dy
    oy = cb * dy - sb * dx
    out_ref[...] = ox * ox + oy * oy


def kernel(gt_bboxes, points):
    num_gts = gt_bboxes.shape[0]
    num_points = points.shape[0]
    gt_t = gt_bboxes.T
    grid = (num_points // _BLK,)
    return pl.pallas_call(
        _body,
        grid=grid,
        in_specs=[
            pl.BlockSpec((5, num_gts), lambda i: (0, 0)),
            pl.BlockSpec((_BLK, 2), lambda i: (i, 0)),
        ],
        out_specs=pl.BlockSpec((_BLK, num_gts), lambda i: (i, 0)),
        out_shape=jax.ShapeDtypeStruct((num_points, num_gts), jnp.float32),
    )(gt_t, points)
